# baseline (speedup 1.0000x reference)
import jax
import jax.numpy as jnp
from jax.experimental import pallas as pl

_NP = 4096
_P = 64
_BM = 32

_DN = (((2,), (2,)), ((0,), (0,)))


def _chamfer_body(pred_ref, tgt_ref, pn_ref, tn_ref, pn1_ref, tn1_ref, out_ref):
    @pl.when(pl.program_id(0) == 0)
    def _init():
        out_ref[...] = jnp.zeros_like(out_ref)

    p = pred_ref[...]
    t = tgt_ref[...]
    pn = pn_ref[...]
    tn = tn_ref[...]
    pn1 = pn1_ref[...]
    tn1 = tn1_ref[...]

    cross = jax.lax.dot_general(p, t, _DN, preferred_element_type=jnp.float32)
    crosst = jax.lax.dot_general(t, p, _DN, preferred_element_type=jnp.float32)

    fwd = pn + jnp.min(tn1 - 2.0 * crosst, axis=1)
    bwd = tn + jnp.min(pn1 - 2.0 * cross, axis=1)
    out_ref[...] += (jnp.sum(fwd) + jnp.sum(bwd)).reshape(1, 1)


def kernel(pred_patches, target_patches):
    pred = pred_patches.reshape(_NP, _P, 3)
    tgt = target_patches.reshape(_NP, _P, 3)
    pn = jnp.sum(pred * pred, axis=2)
    tn = jnp.sum(tgt * tgt, axis=2)

    raw = pl.BlockSpec((_BM, _P, 3), lambda i: (i, 0, 0))
    plane = pl.BlockSpec((_BM, _P), lambda i: (i, 0))
    col = pl.BlockSpec((_BM, _P, 1), lambda i: (i, 0, 0))
    total = pl.pallas_call(
        _chamfer_body,
        grid=(_NP // _BM,),
        in_specs=[raw, raw, plane, plane, col, col],
        out_specs=pl.BlockSpec((1, 1), lambda i: (0, 0)),
        out_shape=jax.ShapeDtypeStruct((1, 1), jnp.float32),
    )(pred, tgt, pn, tn, pn.reshape(_NP, _P, 1), tn.reshape(_NP, _P, 1))

    return total[0, 0] * (1.0 / (_NP * _P))


# speedup vs baseline: 2.4731x; 2.4731x over previous
import jax
import jax.numpy as jnp
from jax.experimental import pallas as pl

_NP = 4096
_P = 64
_BM = 64

_DN = (((2,), (2,)), ((0,), (0,)))


def _chamfer_body(pred_ref, tgt_ref, out_ref):
    @pl.when(pl.program_id(0) == 0)
    def _init():
        out_ref[...] = jnp.zeros_like(out_ref)

    p = pred_ref[...]
    t = tgt_ref[...]
    pm2 = -2.0 * p
    pn1 = jnp.sum(p * p, axis=2, keepdims=True)
    tn1 = jnp.sum(t * t, axis=2, keepdims=True)

    e_bwd = pn1 + jax.lax.dot_general(pm2, t, _DN, preferred_element_type=jnp.float32)
    e_fwd = tn1 + jax.lax.dot_general(t, pm2, _DN, preferred_element_type=jnp.float32)

    mb = jnp.min(e_bwd, axis=1)
    mf = jnp.min(e_fwd, axis=1)
    step = jnp.sum(mb) + jnp.sum(mf) + jnp.sum(pn1) + jnp.sum(tn1)
    out_ref[...] += step.reshape(1, 1)


def kernel(pred_patches, target_patches):
    pred = pred_patches.reshape(_NP, _P, 3)
    tgt = target_patches.reshape(_NP, _P, 3)

    raw = pl.BlockSpec((_BM, _P, 3), lambda i: (i, 0, 0))
    total = pl.pallas_call(
        _chamfer_body,
        grid=(_NP // _BM,),
        in_specs=[raw, raw],
        out_specs=pl.BlockSpec((1, 1), lambda i: (0, 0)),
        out_shape=jax.ShapeDtypeStruct((1, 1), jnp.float32),
    )(pred, tgt)

    return total[0, 0] * (1.0 / (_NP * _P))


# speedup vs baseline: 5.7869x; 2.3399x over previous
import jax
import jax.numpy as jnp
from jax.experimental import pallas as pl

_NP = 4096
_P = 64
_BM = 64

_DN = (((1,), (1,)), ((0,), (0,)))


def _chamfer_body(pred_ref, tgt_ref, out_ref):
    @pl.when(pl.program_id(0) == 0)
    def _init():
        out_ref[...] = jnp.zeros_like(out_ref)

    p = pred_ref[...]
    t = tgt_ref[...]
    pn = jnp.sum(p * p, axis=1, keepdims=True)
    tn = jnp.sum(t * t, axis=1, keepdims=True)
    ones = jnp.ones_like(pn)

    lhs_p = jnp.concatenate([p, pn], axis=1)
    lhs_t = jnp.concatenate([t, tn], axis=1)
    rhs_p = jnp.concatenate([-2.0 * p, ones], axis=1)
    rhs_t = jnp.concatenate([-2.0 * t, ones], axis=1)

    e_bwd = jax.lax.dot_general(lhs_p, rhs_t, _DN, preferred_element_type=jnp.float32)
    e_fwd = jax.lax.dot_general(lhs_t, rhs_p, _DN, preferred_element_type=jnp.float32)

    mb = jnp.min(e_bwd, axis=1)
    mf = jnp.min(e_fwd, axis=1)
    step = jnp.sum(mb) + jnp.sum(mf) + jnp.sum(pn) + jnp.sum(tn)
    out_ref[...] += step.reshape(1, 1)


def kernel(pred_patches, target_patches):
    pred = pred_patches.reshape(_NP, _P, 3).swapaxes(1, 2)
    tgt = target_patches.reshape(_NP, _P, 3).swapaxes(1, 2)

    raw = pl.BlockSpec((_BM, 3, _P), lambda i: (i, 0, 0))
    total = pl.pallas_call(
        _chamfer_body,
        grid=(_NP // _BM,),
        in_specs=[raw, raw],
        out_specs=pl.BlockSpec((1, 1), lambda i: (0, 0)),
        out_shape=jax.ShapeDtypeStruct((1, 1), jnp.float32),
    )(pred, tgt)

    return total[0, 0] * (1.0 / (_NP * _P))


# speedup vs baseline: 6.1963x; 1.0708x over previous
import jax
import jax.numpy as jnp
from jax.experimental import pallas as pl

_NP = 4096
_P = 64
_BM = 64

_DN = (((1,), (1,)), ((0,), (0,)))


def _chamfer_body(pred_ref, tgt_ref, out_ref):
    @pl.when(pl.program_id(0) == 0)
    def _init():
        out_ref[...] = jnp.zeros_like(out_ref)

    p = pred_ref[...]
    t = tgt_ref[...]
    pn = jnp.sum(p * p, axis=1, keepdims=True)
    tn = jnp.sum(t * t, axis=1, keepdims=True)
    ones = jnp.ones_like(pn)

    lhs = jnp.concatenate([p, pn, ones], axis=1)
    rhs = jnp.concatenate([-2.0 * t, ones, tn], axis=1)

    d2 = jax.lax.dot_general(lhs, rhs, _DN, preferred_element_type=jnp.float32)

    fwd = jnp.min(d2, axis=2)
    bwd = jnp.min(d2, axis=1)
    step = jnp.sum(fwd) + jnp.sum(bwd)
    out_ref[...] += step.reshape(1, 1)


def kernel(pred_patches, target_patches):
    pred = pred_patches.reshape(_NP, _P, 3).swapaxes(1, 2)
    tgt = target_patches.reshape(_NP, _P, 3).swapaxes(1, 2)

    raw = pl.BlockSpec((_BM, 3, _P), lambda i: (i, 0, 0))
    total = pl.pallas_call(
        _chamfer_body,
        grid=(_NP // _BM,),
        in_specs=[raw, raw],
        out_specs=pl.BlockSpec((1, 1), lambda i: (0, 0)),
        out_shape=jax.ShapeDtypeStruct((1, 1), jnp.float32),
    )(pred, tgt)

    return total[0, 0] * (1.0 / (_NP * _P))
